# baseline (speedup 1.0000x reference)
import jax
import jax.numpy as jnp
from jax.experimental import pallas as pl
from jax.experimental.pallas import tpu as pltpu


def kernel(predictions, image_grids, target_boxes_grid):
    B, H, W = predictions.shape
    HW = H * W
    pred2 = predictions.reshape(B, HW)
    grids2 = image_grids.reshape((H + 1) * (W + 1), B, HW)
    gi = H * (W + 1) + W

    def body(x_ref, g_ref, o_ref):
        x = x_ref[...]
        m = g_ref[0].astype(jnp.float32)
        s_pm = jnp.sum(x * m, axis=1, keepdims=True)
        cnt = jnp.sum(m, axis=1, keepdims=True)
        s_p = jnp.sum(x, axis=1, keepdims=True)
        mean_t = s_pm / cnt
        lt = -jnp.log(mean_t)
        mean_b = (s_p - s_pm) / (HW - cnt)
        lb = jnp.nan_to_num(-jnp.log(1.0 - mean_b))
        o_ref[...] = ((jnp.sum(lb) + jnp.sum(lt)) / B).reshape(1, 1)

    out = pl.pallas_call(
        body,
        grid=(1,),
        in_specs=[
            pl.BlockSpec((B, HW), lambda i: (0, 0)),
            pl.BlockSpec((1, B, HW), lambda i: (gi, 0, 0)),
        ],
        out_specs=pl.BlockSpec((1, 1), lambda i: (0, 0)),
        out_shape=jax.ShapeDtypeStruct((1, 1), jnp.float32),
    )(pred2, grids2)
    return out[0, 0]
